# baseline (speedup 1.0000x reference)
import functools

import numpy as np
import jax
import jax.numpy as jnp
from jax import lax
from jax.experimental import pallas as pl
from jax.experimental.pallas import tpu as pltpu
from jax.experimental.pallas import tpu_sc as plsc

WINDOW_SIZE = 100000
E = 64
B = 4096
S = 200
SCALE = 8.0

NC = 2
NS = 16
NW = NC * NS
SEQ_PER_W = B // NW
CHUNK = 40
KPS = S // CHUNK
NCHUNK = SEQ_PER_W * KPS
NBUF = 4


def _positional_encoding() -> np.ndarray:
    half = E // 2
    positions = np.arange(S, dtype=np.float32)[:, None]
    depths = np.arange(half, dtype=np.float32)[None, :] / float(half)
    angle_rads = positions * (1.0 / (10000.0 ** depths))
    return np.concatenate(
        [np.sin(angle_rads), np.cos(angle_rads)], axis=-1
    ).astype(np.float32)


_POS = _positional_encoding()


_MESH = plsc.VectorSubcoreMesh(core_axis_name="c", subcore_axis_name="s")


@functools.partial(
    pl.kernel,
    mesh=_MESH,
    compiler_params=pltpu.CompilerParams(use_tc_tiling_on_sc=False),
    out_type=jax.ShapeDtypeStruct((B, S, E), jnp.float32),
    scratch_types=[
        pltpu.VMEM((SEQ_PER_W, S), jnp.int32),
        pltpu.VMEM((S, E), jnp.float32),
    ]
    + [pltpu.VMEM((CHUNK, E), jnp.float32) for _ in range(NBUF)]
    + [pltpu.SemaphoreType.DMA for _ in range(2 * NBUF)],
)
def _embed_pos(x_hbm, table_hbm, pos_hbm, out_hbm, idx_v, pos_v, *bufs_sems):
    rbufs = bufs_sems[:NBUF]
    gsems = bufs_sems[NBUF : 2 * NBUF]
    ssems = bufs_sems[2 * NBUF :]
    wid = lax.axis_index("s") * NC + lax.axis_index("c")
    seq0 = wid * SEQ_PER_W
    pltpu.sync_copy(x_hbm.at[pl.ds(seq0, SEQ_PER_W)], idx_v)
    pltpu.sync_copy(pos_hbm, pos_v)

    def gather_start(c, b):
        q = lax.div(c, KPS)
        k = lax.rem(c, KPS)
        pltpu.async_copy(
            table_hbm.at[idx_v.at[q, pl.ds(k * CHUNK, CHUNK)]],
            rbufs[b],
            gsems[b],
        )

    def store_start(c, b):
        q = lax.div(c, KPS)
        k = lax.rem(c, KPS)
        pltpu.async_copy(
            rbufs[b], out_hbm.at[seq0 + q, pl.ds(k * CHUNK, CHUNK)], ssems[b]
        )

    for b in range(NBUF):
        gather_start(b, b)

    def outer(c0, carry):
        for b in range(NBUF):
            c = c0 * NBUF + b
            rows = rbufs[b]
            pltpu.make_async_copy(
                table_hbm.at[idx_v.at[0, pl.ds(0, CHUNK)]], rows, gsems[b]
            ).wait()
            off = lax.rem(c, KPS) * CHUNK

            def s_body(s, carry2):
                for j in range(E // 16):
                    sl = pl.ds(j * 16, 16)
                    rows[s, sl] = rows[s, sl] * SCALE + pos_v[off + s, sl]
                return carry2

            lax.fori_loop(0, CHUNK, s_body, 0, unroll=2)

            pb = (b - 1) % NBUF

            @pl.when(c > 0)
            def _():
                pltpu.make_async_copy(
                    rbufs[pb], out_hbm.at[0, pl.ds(0, CHUNK)], ssems[pb]
                ).wait()

            @pl.when((c > 0) & (c - 1 + NBUF < NCHUNK))
            def _():
                gather_start(c - 1 + NBUF, pb)

            store_start(c, b)
        return carry

    lax.fori_loop(0, NCHUNK // NBUF, outer, 0)
    lb = (NCHUNK - 1) % NBUF
    pltpu.make_async_copy(
        rbufs[lb], out_hbm.at[0, pl.ds(0, CHUNK)], ssems[lb]
    ).wait()


@jax.jit
def _run(x, table):
    return _embed_pos(x.astype(jnp.int32), table, jnp.asarray(_POS))


def kernel(x, table):
    return _run(x, table)
